# baseline (speedup 1.0000x reference)
import functools

import jax
import jax.numpy as jnp
from jax import lax
from jax.experimental import pallas as pl
from jax.experimental.pallas import tpu as pltpu
from jax.experimental.pallas import tpu_sc as plsc

VOCAB = 100000
EMBED = 64
CHORD_START = 1000
CHORD_END = 5000
B, S = 4096, 50

TOKS = B * S
NC, NS, L = 2, 16, 16
NW = NC * NS
TPW = TOKS // NW
CHUNK = 128
NCHUNK = TPW // CHUNK

ADJ_ROWS = 4096
COMBO_BASE = ADJ_ROWS
ZROW = COMBO_BASE + 128
AUX_ROWS = ZROW + 8


def _tc_precompute_body(tt_ref, w1_ref, w2_ref, w3_ref, b_ref, root_ref,
                        type_ref, out_ref):
    tt = tt_ref[:]
    proj = jax.lax.dot_general(tt, w1_ref[:], (((1,), (1,)), ((), ())),
                               preferred_element_type=jnp.float32)
    out_ref[0:ADJ_ROWS, :] = proj + b_ref[:] - tt
    rp = jax.lax.dot_general(root_ref[:], w2_ref[:], (((1,), (1,)), ((), ())),
                             preferred_element_type=jnp.float32)
    tp = jax.lax.dot_general(type_ref[:], w3_ref[:], (((1,), (1,)), ((), ())),
                             preferred_element_type=jnp.float32)
    for r in range(16):
        out_ref[COMBO_BASE + 8 * r:COMBO_BASE + 8 * r + 8, :] = rp[r:r + 1, :] + tp
    out_ref[ZROW:AUX_ROWS, :] = jnp.zeros((AUX_ROWS - ZROW, EMBED), jnp.float32)


_tc_precompute = pl.pallas_call(
    _tc_precompute_body,
    out_shape=jax.ShapeDtypeStruct((AUX_ROWS, EMBED), jnp.float32),
)


def _sc_main_body(ids_hbm, roots_hbm, types_hbm, table_hbm, aux_hbm, out_hbm,
                  ids_v, roots_v, types_v, idx2_v, idx3_v,
                  buf_a, buf_b, buf_c, sem_a, sem_b, sem_c):
    wid = lax.axis_index("s") * NC + lax.axis_index("c")
    row0 = wid * NCHUNK

    pltpu.sync_copy(ids_hbm.at[wid], ids_v)
    pltpu.sync_copy(roots_hbm.at[wid], roots_v)
    pltpu.sync_copy(types_hbm.at[wid], types_v)

    zrow = jnp.full((L,), ZROW, jnp.int32)

    def compute_indices(j, carry):
        for i in range(CHUNK // L):
            sl = pl.ds(i * L, L)
            tid = ids_v[j, sl]
            rid = roots_v[j, sl]
            cid = types_v[j, sl]
            isch = (tid >= CHORD_START) & (tid <= CHORD_END)
            idx2_v[j, sl] = jnp.where(isch, tid - CHORD_START, zrow)
            idx3_v[j, sl] = jnp.where(isch, COMBO_BASE + rid * 8 + cid, zrow)
        return carry

    lax.fori_loop(0, NCHUNK, compute_indices, 0)

    def do_chunk(j, carry):
        cp_a = pltpu.async_copy(table_hbm.at[ids_v.at[j]], buf_a, sem_a)
        cp_b = pltpu.async_copy(aux_hbm.at[idx2_v.at[j]], buf_b, sem_b)
        cp_c = pltpu.async_copy(aux_hbm.at[idx3_v.at[j]], buf_c, sem_c)
        cp_a.wait()
        cp_b.wait()
        cp_c.wait()

        def combine(r, inner):
            for c in range(EMBED // L):
                sl = pl.ds(c * L, L)
                plsc.addupdate(buf_a.at[r, sl], buf_b[r, sl] + buf_c[r, sl])
            return inner

        lax.fori_loop(0, CHUNK, combine, 0)
        pltpu.sync_copy(buf_a, out_hbm.at[pl.ds((row0 + j) * CHUNK, CHUNK)])
        return carry

    lax.fori_loop(0, NCHUNK, do_chunk, 0)


_sc_main = functools.partial(
    pl.kernel,
    out_type=jax.ShapeDtypeStruct((TOKS, EMBED), jnp.float32),
    mesh=plsc.VectorSubcoreMesh(core_axis_name="c", subcore_axis_name="s"),
    compiler_params=pltpu.CompilerParams(use_tc_tiling_on_sc=False),
    scratch_types=[
        pltpu.VMEM((NCHUNK, CHUNK), jnp.int32),
        pltpu.VMEM((NCHUNK, CHUNK), jnp.int32),
        pltpu.VMEM((NCHUNK, CHUNK), jnp.int32),
        pltpu.VMEM((NCHUNK, CHUNK), jnp.int32),
        pltpu.VMEM((NCHUNK, CHUNK), jnp.int32),
        pltpu.VMEM((CHUNK, EMBED), jnp.float32),
        pltpu.VMEM((CHUNK, EMBED), jnp.float32),
        pltpu.VMEM((CHUNK, EMBED), jnp.float32),
        pltpu.SemaphoreType.DMA,
        pltpu.SemaphoreType.DMA,
        pltpu.SemaphoreType.DMA,
    ],
)(_sc_main_body)


def kernel(token_ids, chord_root_ids, chord_type_ids, token_table, root_table,
           type_table, W, b):
    ids2d = token_ids.astype(jnp.int32).reshape(NW, NCHUNK, CHUNK)
    roots2d = chord_root_ids.astype(jnp.int32).reshape(NW, NCHUNK, CHUNK)
    types2d = chord_type_ids.astype(jnp.int32).reshape(NW, NCHUNK, CHUNK)

    tt_chord = lax.slice(token_table, (CHORD_START, 0),
                         (CHORD_START + ADJ_ROWS, EMBED))
    w1 = lax.slice(W, (0, 0), (EMBED, EMBED))
    w2 = lax.slice(W, (0, EMBED), (EMBED, 2 * EMBED))
    w3 = lax.slice(W, (0, 2 * EMBED), (EMBED, 3 * EMBED))
    root_pad = jnp.pad(root_table, ((0, 16 - root_table.shape[0]), (0, 0)))

    aux = _tc_precompute(tt_chord, w1, w2, w3, b.reshape(1, EMBED), root_pad,
                         type_table)
    out = _sc_main(ids2d, roots2d, types2d, token_table, aux)
    return out.reshape(B, S, EMBED)


# speedup vs baseline: 1.8883x; 1.8883x over previous
import functools

import jax
import jax.numpy as jnp
from jax import lax
from jax.experimental import pallas as pl
from jax.experimental.pallas import tpu as pltpu
from jax.experimental.pallas import tpu_sc as plsc

VOCAB = 100000
EMBED = 64
CHORD_START = 1000
CHORD_END = 5000
B, S = 4096, 50

TOKS = B * S
NC, NS, L = 2, 16, 16
NW = NC * NS
TPW = TOKS // NW
CHUNK = 128
CWORDS = CHUNK * EMBED
NCHUNK = TPW // CHUNK
NBUF = 5
NROUND = NCHUNK // NBUF

COMBO_ZROW = 104
COMBO_ROWS = 112

FUSE_BLK = 4000


def _tc_fuse_body(tt_ref, w1_ref, b_ref, out_ref):
    i = pl.program_id(0)
    tt = tt_ref[:]
    rows = jax.lax.broadcasted_iota(jnp.int32, (FUSE_BLK, EMBED), 0)
    rows = rows + i * FUSE_BLK
    is_chord = (rows >= CHORD_START) & (rows <= CHORD_END)
    proj = jax.lax.dot_general(tt, w1_ref[:], (((1,), (1,)), ((), ())),
                               preferred_element_type=jnp.float32)
    out_ref[:] = jnp.where(is_chord, proj + b_ref[:], tt)


_tc_fuse = pl.pallas_call(
    _tc_fuse_body,
    grid=(VOCAB // FUSE_BLK,),
    in_specs=[
        pl.BlockSpec((FUSE_BLK, EMBED), lambda i: (i, 0)),
        pl.BlockSpec((EMBED, EMBED), lambda i: (0, 0)),
        pl.BlockSpec((1, EMBED), lambda i: (0, 0)),
    ],
    out_specs=pl.BlockSpec((FUSE_BLK, EMBED), lambda i: (i, 0)),
    out_shape=jax.ShapeDtypeStruct((VOCAB, EMBED), jnp.float32),
)


def _tc_combo_body(root_ref, type_ref, w2_ref, w3_ref, out_ref):
    rp = jax.lax.dot_general(root_ref[:], w2_ref[:], (((1,), (1,)), ((), ())),
                             preferred_element_type=jnp.float32)
    tp = jax.lax.dot_general(type_ref[:], w3_ref[:], (((1,), (1,)), ((), ())),
                             preferred_element_type=jnp.float32)
    for r in range(13):
        out_ref[8 * r:8 * r + 8, :] = rp[r:r + 1, :] + tp
    out_ref[COMBO_ZROW:COMBO_ROWS, :] = jnp.zeros(
        (COMBO_ROWS - COMBO_ZROW, EMBED), jnp.float32)


_tc_combo = pl.pallas_call(
    _tc_combo_body,
    out_shape=jax.ShapeDtypeStruct((COMBO_ROWS, EMBED), jnp.float32),
)


def _sc_main_body(ids_hbm, roots_hbm, types_hbm, fused_hbm, combo_hbm, out_hbm,
                  ids_v, roots_v, types_v, cidx_v, buf, cbuf,
                  gsem, csem, ssem):
    wid = lax.axis_index("s") * NC + lax.axis_index("c")

    pltpu.sync_copy(ids_hbm.at[wid], ids_v)
    pltpu.sync_copy(roots_hbm.at[wid], roots_v)
    pltpu.sync_copy(types_hbm.at[wid], types_v)

    zrow = jnp.full((L,), COMBO_ZROW, jnp.int32)

    def compute_indices(j, carry):
        for i in range(CHUNK // L):
            sl = pl.ds(i * L, L)
            tid = ids_v[j, sl]
            isch = (tid >= CHORD_START) & (tid <= CHORD_END)
            cidx_v[j, sl] = jnp.where(
                isch, roots_v[j, sl] * 8 + types_v[j, sl], zrow)
        return carry

    lax.fori_loop(0, NCHUNK, compute_indices, 0)

    def start_chunk(j, bslot):
        pltpu.async_copy(fused_hbm.at[ids_v.at[j]], buf.at[bslot],
                         gsem.at[bslot])
        pltpu.async_copy(combo_hbm.at[cidx_v.at[j]], cbuf.at[bslot],
                         csem.at[bslot])

    for bslot in range(NBUF):
        start_chunk(bslot, bslot)

    def do_round(r, carry):
        for bslot in range(NBUF):
            j = r * NBUF + bslot
            bb = buf.at[bslot]
            cb = cbuf.at[bslot]
            pltpu.make_async_copy(fused_hbm.at[ids_v.at[j]], bb,
                                  gsem.at[bslot]).wait()
            pltpu.make_async_copy(combo_hbm.at[cidx_v.at[j]], cb,
                                  csem.at[bslot]).wait()

            def combine(t, inner):
                for c in range(EMBED // L):
                    sl = pl.ds(c * L, L)
                    plsc.addupdate(bb.at[t, sl], cb[t, sl])
                return inner

            lax.fori_loop(0, CHUNK, combine, 0)

            @pl.when(r > 0)
            def _drain():
                pltpu.make_async_copy(
                    bb, out_hbm.at[pl.ds((wid * NCHUNK + j - NBUF) * CHUNK,
                                         CHUNK)], ssem.at[bslot]).wait()

            pltpu.async_copy(bb, out_hbm.at[pl.ds((wid * NCHUNK + j) * CHUNK,
                                                  CHUNK)], ssem.at[bslot])

            @pl.when(r < NROUND - 1)
            def _prefetch():
                start_chunk(j + NBUF, bslot)
        return carry

    lax.fori_loop(0, NROUND, do_round, 0)

    for bslot in range(NBUF):
        j = (NROUND - 1) * NBUF + bslot
        pltpu.make_async_copy(
            buf.at[bslot],
            out_hbm.at[pl.ds((wid * NCHUNK + j) * CHUNK, CHUNK)],
            ssem.at[bslot]).wait()


_sc_main = functools.partial(
    pl.kernel,
    out_type=jax.ShapeDtypeStruct((TOKS, EMBED), jnp.float32),
    mesh=plsc.VectorSubcoreMesh(core_axis_name="c", subcore_axis_name="s"),
    compiler_params=pltpu.CompilerParams(use_tc_tiling_on_sc=False),
    scratch_types=[
        pltpu.VMEM((NCHUNK, CHUNK), jnp.int32),
        pltpu.VMEM((NCHUNK, CHUNK), jnp.int32),
        pltpu.VMEM((NCHUNK, CHUNK), jnp.int32),
        pltpu.VMEM((NCHUNK, CHUNK), jnp.int32),
        pltpu.VMEM((NBUF, CHUNK, EMBED), jnp.float32),
        pltpu.VMEM((NBUF, CHUNK, EMBED), jnp.float32),
        pltpu.SemaphoreType.DMA((NBUF,)),
        pltpu.SemaphoreType.DMA((NBUF,)),
        pltpu.SemaphoreType.DMA((NBUF,)),
    ],
)(_sc_main_body)


def kernel(token_ids, chord_root_ids, chord_type_ids, token_table, root_table,
           type_table, W, b):
    ids3d = token_ids.astype(jnp.int32).reshape(NW, NCHUNK, CHUNK)
    roots3d = chord_root_ids.astype(jnp.int32).reshape(NW, NCHUNK, CHUNK)
    types3d = chord_type_ids.astype(jnp.int32).reshape(NW, NCHUNK, CHUNK)

    w1 = lax.slice(W, (0, 0), (EMBED, EMBED))
    w2 = lax.slice(W, (0, EMBED), (EMBED, 2 * EMBED))
    w3 = lax.slice(W, (0, 2 * EMBED), (EMBED, 3 * EMBED))
    root_pad = jnp.pad(root_table, ((0, 16 - root_table.shape[0]), (0, 0)))

    fused = _tc_fuse(token_table, w1, b.reshape(1, EMBED))
    combo = _tc_combo(root_pad, type_table, w2, w3)
    out = _sc_main(ids3d, roots3d, types3d, fused, combo)
    return out.reshape(B, S, EMBED)


# speedup vs baseline: 15.9293x; 8.4357x over previous
import functools

import jax
import jax.numpy as jnp
from jax import lax
from jax.experimental import pallas as pl
from jax.experimental.pallas import tpu as pltpu
from jax.experimental.pallas import tpu_sc as plsc

VOCAB = 100000
EMBED = 64
CHORD_START = 1000
CHORD_END = 5000
B, S = 4096, 50

TOKS = B * S
NC, NS, L = 2, 16, 16
NW = NC * NS
TPW = TOKS // NW
CHUNK = 128
NCHUNK = TPW // CHUNK
NBUF = 10
NROUND = NCHUNK // NBUF

COMBO_ZROW = 104
COMBO_ROWS = 112

FUSE_BLK = 4000
POST_ROWS = 32


def _tc_fuse_body(tt_ref, w1_ref, b_ref, out_ref):
    i = pl.program_id(0)
    tt = tt_ref[:]
    rows = jax.lax.broadcasted_iota(jnp.int32, (FUSE_BLK, EMBED), 0)
    rows = rows + i * FUSE_BLK
    is_chord = (rows >= CHORD_START) & (rows <= CHORD_END)
    proj = jax.lax.dot_general(tt, w1_ref[:], (((1,), (1,)), ((), ())),
                               preferred_element_type=jnp.float32)
    out_ref[:] = jnp.where(is_chord, proj + b_ref[:], tt)


_tc_fuse = pl.pallas_call(
    _tc_fuse_body,
    grid=(VOCAB // FUSE_BLK,),
    in_specs=[
        pl.BlockSpec((FUSE_BLK, EMBED), lambda i: (i, 0)),
        pl.BlockSpec((EMBED, EMBED), lambda i: (0, 0)),
        pl.BlockSpec((1, EMBED), lambda i: (0, 0)),
    ],
    out_specs=pl.BlockSpec((FUSE_BLK, EMBED), lambda i: (i, 0)),
    out_shape=jax.ShapeDtypeStruct((VOCAB, EMBED), jnp.float32),
)


def _tc_combo_body(root_ref, type_ref, w2_ref, w3_ref, out_ref):
    rp = jax.lax.dot_general(root_ref[:], w2_ref[:], (((1,), (1,)), ((), ())),
                             preferred_element_type=jnp.float32)
    tp = jax.lax.dot_general(type_ref[:], w3_ref[:], (((1,), (1,)), ((), ())),
                             preferred_element_type=jnp.float32)
    for r in range(13):
        out_ref[8 * r:8 * r + 8, :] = rp[r:r + 1, :] + tp
    out_ref[COMBO_ZROW:COMBO_ROWS, :] = jnp.zeros(
        (COMBO_ROWS - COMBO_ZROW, EMBED), jnp.float32)


_tc_combo = pl.pallas_call(
    _tc_combo_body,
    out_shape=jax.ShapeDtypeStruct((COMBO_ROWS, EMBED), jnp.float32),
)


def _sc_gather_body(ids_hbm, fused_hbm, out_hbm, ids_v, buf, gsem, ssem):
    wid = lax.axis_index("s") * NC + lax.axis_index("c")

    pltpu.sync_copy(ids_hbm.at[wid], ids_v)

    for bslot in range(NBUF):
        pltpu.async_copy(fused_hbm.at[ids_v.at[bslot]], buf.at[bslot],
                         gsem.at[bslot])

    def do_round(r, carry):
        for bslot in range(NBUF):
            j = r * NBUF + bslot
            bb = buf.at[bslot]
            pltpu.make_async_copy(fused_hbm.at[ids_v.at[j]], bb,
                                  gsem.at[bslot]).wait()

            @pl.when(r > 0)
            def _drain():
                pltpu.make_async_copy(
                    bb, out_hbm.at[pl.ds((wid * NCHUNK + j - NBUF) * CHUNK,
                                         CHUNK)], ssem.at[bslot]).wait()

            pltpu.async_copy(bb, out_hbm.at[pl.ds((wid * NCHUNK + j) * CHUNK,
                                                  CHUNK)], ssem.at[bslot])

            @pl.when(r < NROUND - 1)
            def _prefetch():
                pltpu.async_copy(fused_hbm.at[ids_v.at[j + NBUF]], bb,
                                 gsem.at[bslot])
        return carry

    lax.fori_loop(0, NROUND, do_round, 0)

    for bslot in range(NBUF):
        j = (NROUND - 1) * NBUF + bslot
        pltpu.make_async_copy(
            buf.at[bslot],
            out_hbm.at[pl.ds((wid * NCHUNK + j) * CHUNK, CHUNK)],
            ssem.at[bslot]).wait()


_sc_gather = functools.partial(
    pl.kernel,
    out_type=jax.ShapeDtypeStruct((TOKS, EMBED), jnp.float32),
    mesh=plsc.VectorSubcoreMesh(core_axis_name="c", subcore_axis_name="s"),
    compiler_params=pltpu.CompilerParams(use_tc_tiling_on_sc=False),
    scratch_types=[
        pltpu.VMEM((NCHUNK, CHUNK), jnp.int32),
        pltpu.VMEM((NBUF, CHUNK, EMBED), jnp.float32),
        pltpu.SemaphoreType.DMA((NBUF,)),
        pltpu.SemaphoreType.DMA((NBUF,)),
    ],
)(_sc_gather_body)


def _tc_post_body(rows_ref, ids_ref, roots_ref, types_ref, combo_ref, out_ref):
    tid = ids_ref[:]
    is_chord = (tid >= CHORD_START) & (tid <= CHORD_END)
    cidx = jnp.where(is_chord, roots_ref[:] * 8 + types_ref[:], COMBO_ZROW)
    kidx = jax.lax.broadcasted_iota(jnp.int32, (POST_ROWS, 128, COMBO_ROWS), 2)
    one_hot = (cidx[:, :, None] == kidx).astype(jnp.float32)
    contrib = jax.lax.dot_general(
        one_hot, combo_ref[:], (((2,), (0,)), ((), ())),
        preferred_element_type=jnp.float32)
    out_ref[:] = rows_ref[:] + contrib


_tc_post = pl.pallas_call(
    _tc_post_body,
    grid=(TOKS // (POST_ROWS * 128),),
    in_specs=[
        pl.BlockSpec((POST_ROWS, 128, EMBED), lambda i: (i, 0, 0)),
        pl.BlockSpec((POST_ROWS, 128), lambda i: (i, 0)),
        pl.BlockSpec((POST_ROWS, 128), lambda i: (i, 0)),
        pl.BlockSpec((POST_ROWS, 128), lambda i: (i, 0)),
        pl.BlockSpec((COMBO_ROWS, EMBED), lambda i: (0, 0)),
    ],
    out_specs=pl.BlockSpec((POST_ROWS, 128, EMBED), lambda i: (i, 0, 0)),
    out_shape=jax.ShapeDtypeStruct((TOKS // 128, 128, EMBED), jnp.float32),
)


def kernel(token_ids, chord_root_ids, chord_type_ids, token_table, root_table,
           type_table, W, b):
    ids3d = token_ids.astype(jnp.int32).reshape(NW, NCHUNK, CHUNK)
    ids2d = token_ids.astype(jnp.int32).reshape(TOKS // 128, 128)
    roots2d = chord_root_ids.astype(jnp.int32).reshape(TOKS // 128, 128)
    types2d = chord_type_ids.astype(jnp.int32).reshape(TOKS // 128, 128)

    w1 = lax.slice(W, (0, 0), (EMBED, EMBED))
    w2 = lax.slice(W, (0, EMBED), (EMBED, 2 * EMBED))
    w3 = lax.slice(W, (0, 2 * EMBED), (EMBED, 3 * EMBED))
    root_pad = jnp.pad(root_table, ((0, 16 - root_table.shape[0]), (0, 0)))

    fused = _tc_fuse(token_table, w1, b.reshape(1, EMBED))
    combo = _tc_combo(root_pad, type_table, w2, w3)
    rows = _sc_gather(ids3d, fused).reshape(TOKS // 128, 128, EMBED)
    out = _tc_post(rows, ids2d, roots2d, types2d, combo)
    return out.reshape(B, S, EMBED)
